# baseline (speedup 1.0000x reference)
import functools

import jax
import jax.numpy as jnp
from jax import lax
from jax.experimental import pallas as pl
from jax.experimental.pallas import tpu as pltpu
from jax.experimental.pallas import tpu_sc as plsc

N = 10000
E = 320000
D = 128
DH = D // 2
NC, NS, L = 2, 16, 16
NW = NC * NS
K = 128
CW = 80
EP = NW * CW * K
KR = 64
CWF = 320
NACC = 10240
RPT = NACC // NS
NACC_R = 10112
RPT_R = NACC_R // NS
DUMMY = N
RB = 400
G = N // RB

_mesh = plsc.VectorSubcoreMesh(core_axis_name="c", subcore_axis_name="s")
NBUF = 4
RBUF = 2
GG = 16


@functools.partial(
    pl.kernel,
    out_type=jax.ShapeDtypeStruct((NC, NACC), jnp.float32),
    mesh=_mesh,
    scratch_types=[
        pltpu.VMEM((CW, K), jnp.int32),
        pltpu.VMEM((K,), jnp.float32),
        pltpu.VMEM_SHARED((NACC,), jnp.float32),
        pltpu.SemaphoreType.DMA,
    ],
)
def _sc_deg(colp_hbm, z1_hbm, out_hbm, colb, oneb, acc, sem):
    cid = lax.axis_index("c")
    sid = lax.axis_index("s")
    wid = sid * NC + cid
    pltpu.sync_copy(z1_hbm.at[pl.ds(sid * RPT, RPT)],
                    acc.at[pl.ds(sid * RPT, RPT)])
    pltpu.sync_copy(colp_hbm.at[wid], colb)
    for t in range(K // L):
        oneb[pl.ds(t * L, L)] = jnp.ones((L,), jnp.float32)
    plsc.subcore_barrier()

    def fire(j, carry):
        pltpu.async_copy(oneb, acc.at[colb.at[j]], sem, add=True)
        return carry

    lax.fori_loop(0, CW, fire, 0)

    def drain(j, carry):
        pltpu.make_async_copy(oneb, acc.at[colb.at[j]], sem).wait()
        return carry

    lax.fori_loop(0, CW, drain, 0)
    plsc.subcore_barrier()
    pltpu.sync_copy(acc.at[pl.ds(sid * RPT, RPT)],
                    out_hbm.at[cid, pl.ds(sid * RPT, RPT)])


@functools.partial(
    pl.kernel,
    out_type=jax.ShapeDtypeStruct((NC, NACC), jnp.float32),
    mesh=_mesh,
    scratch_types=[
        pltpu.VMEM((CW, K), jnp.int32),
        pltpu.VMEM((CW, K), jnp.int32),
        pltpu.VMEM((CW, K), jnp.float32),
        pltpu.SemaphoreType.DMA,
        pltpu.VMEM_SHARED((NACC,), jnp.float32),
    ],
)
def _sc_scalar_agg(sp_hbm, rowp_hbm, colp_hbm, z1_hbm, out_hbm,
                   rowb, colb, valarr, sem, acc):
    cid = lax.axis_index("c")
    sid = lax.axis_index("s")
    wid = sid * NC + cid
    pltpu.sync_copy(z1_hbm.at[pl.ds(sid * RPT, RPT)],
                    acc.at[pl.ds(sid * RPT, RPT)])
    pltpu.sync_copy(rowp_hbm.at[wid], rowb)
    pltpu.sync_copy(colp_hbm.at[wid], colb)
    plsc.subcore_barrier()

    def gfire(j, carry):
        pltpu.async_copy(sp_hbm.at[rowb.at[j]], valarr.at[j], sem)
        return carry

    lax.fori_loop(0, CW, gfire, 0)

    def gdrain(j, carry):
        pltpu.make_async_copy(sp_hbm.at[rowb.at[j]], valarr.at[j], sem).wait()
        return carry

    lax.fori_loop(0, CW, gdrain, 0)

    def sfire(j, carry):
        pltpu.async_copy(valarr.at[j], acc.at[colb.at[j]], sem, add=True)
        return carry

    lax.fori_loop(0, CW, sfire, 0)

    def sdrain(j, carry):
        pltpu.make_async_copy(valarr.at[j], acc.at[colb.at[j]], sem).wait()
        return carry

    lax.fori_loop(0, CW, sdrain, 0)
    plsc.subcore_barrier()
    pltpu.sync_copy(acc.at[pl.ds(sid * RPT, RPT)],
                    out_hbm.at[cid, pl.ds(sid * RPT, RPT)])


@functools.partial(
    pl.kernel,
    out_type=jax.ShapeDtypeStruct((NC, NACC, D), jnp.float32),
    mesh=_mesh,
    scratch_types=[
        pltpu.VMEM((CW, K), jnp.int32),
        pltpu.VMEM((GG, K), jnp.int32),
        pltpu.VMEM((K, D), jnp.float32),
        pltpu.VMEM((K, D), jnp.float32),
        pltpu.SemaphoreType.DMA,
        pltpu.SemaphoreType.DMA,
        pltpu.VMEM_SHARED((NACC, D), jnp.float32),
    ],
)
def _sc_row_agg(hp_hbm, rowp_hbm, colp_hbm, z2_hbm, out_hbm,
                rowb, colg, m0, m1, s0, s1, acc):
    msgs = (m0, m1)
    sems = (s0, s1)
    cid = lax.axis_index("c")
    sid = lax.axis_index("s")
    wid = sid * NC + cid
    pltpu.sync_copy(z2_hbm.at[pl.ds(sid * RPT, RPT)],
                    acc.at[pl.ds(sid * RPT, RPT)])
    pltpu.sync_copy(rowp_hbm.at[wid], rowb)
    plsc.subcore_barrier()

    for b in range(RBUF):
        pltpu.async_copy(hp_hbm.at[rowb.at[b]], msgs[b], sems[b])

    def group(g, carry):
        pltpu.sync_copy(colp_hbm.at[wid, pl.ds(g * GG, GG)], colg)

        def step(i, carry2):
            for b in range(RBUF):
                jj = i * RBUF + b
                j = g * GG + jj
                pltpu.make_async_copy(hp_hbm.at[rowb.at[j]], msgs[b],
                                      sems[b]).wait()
                pltpu.sync_copy(msgs[b], acc.at[colg.at[jj]], add=True)
                jn = j + RBUF

                @pl.when(jn < CW)
                def _():
                    pltpu.async_copy(hp_hbm.at[rowb.at[jn]], msgs[b], sems[b])
            return carry2

        lax.fori_loop(0, GG // RBUF, step, 0)
        return carry

    lax.fori_loop(0, CW // GG, group, 0)
    plsc.subcore_barrier()
    pltpu.sync_copy(acc.at[pl.ds(sid * RPT, RPT)],
                    out_hbm.at[cid, pl.ds(sid * RPT, RPT)])


def _tc_prep_body(p0_ref, p1_ref, x_ref, w1_ref, hp_ref, dis_ref):
    deg = 1.0 + p0_ref[0, 0] + p1_ref[0, 0]
    dis = lax.rsqrt(deg)
    h = jnp.dot(x_ref[...], w1_ref[...], preferred_element_type=jnp.float32)
    hp_ref[...] = h * dis[:, None]
    dis_ref[0, 0] = dis


def _tc_mid_body(a0_ref, a1_ref, hp_ref, dis_ref, b1_ref, w2_ref, sp_ref):
    dis = dis_ref[0, 0]
    agg = a0_ref[...] + a1_ref[...] + hp_ref[...]
    out1 = agg * dis[:, None] + b1_ref[...]
    act = jnp.where(out1 >= 0, out1, 0.1 * out1)
    w2bar = jnp.mean(w2_ref[...], axis=1)
    s = jnp.sum(act * w2bar[None, :], axis=1)
    sp_ref[0, 0] = s * dis


def _tc_fin_body(q0_ref, q1_ref, sp_ref, dis_ref, b2_ref, out_ref):
    aggs = q0_ref[0, 0] + q1_ref[0, 0] + sp_ref[0, 0]
    out_ref[0, 0] = dis_ref[0, 0] * aggs + jnp.mean(b2_ref[...])


_blk_rb = pl.BlockSpec((1, 1, RB), lambda i: (i, 0, 0))
_blk_rows = pl.BlockSpec((RB, D), lambda i: (i, 0))
_blk_half = pl.BlockSpec((RB, DH), lambda i: (i, 0))
_blk_full = pl.BlockSpec((D, D), lambda i: (0, 0))
_blk_vec = pl.BlockSpec((1, D), lambda i: (0, 0))

_tc_prep = pl.pallas_call(
    _tc_prep_body,
    grid=(G,),
    in_specs=[_blk_rb, _blk_rb, _blk_rows, _blk_full],
    out_specs=[_blk_rows, _blk_rb],
    out_shape=[
        jax.ShapeDtypeStruct((N, D), jnp.float32),
        jax.ShapeDtypeStruct((G, 1, RB), jnp.float32),
    ],
)

_tc_mid = pl.pallas_call(
    _tc_mid_body,
    grid=(G,),
    in_specs=[_blk_rows, _blk_rows, _blk_rows, _blk_rb, _blk_vec, _blk_full],
    out_specs=[_blk_rb],
    out_shape=[jax.ShapeDtypeStruct((G, 1, RB), jnp.float32)],
)

_tc_fin = pl.pallas_call(
    _tc_fin_body,
    grid=(G,),
    in_specs=[_blk_rb, _blk_rb, _blk_rb, _blk_rb, _blk_vec],
    out_specs=[_blk_rb],
    out_shape=[jax.ShapeDtypeStruct((G, 1, RB), jnp.float32)],
)


def kernel(x, edge_index, W1, b1, W2, b2):
    row = edge_index[0]
    col = edge_index[1]
    row_p = jnp.concatenate([row, jnp.zeros((EP - E,), jnp.int32)])
    col_p = jnp.concatenate([col, jnp.full((EP - E,), DUMMY, jnp.int32)])
    rowp = row_p.reshape(NW, CW, K)
    colp = col_p.reshape(NW, CW, K)
    z1 = jnp.zeros((NACC,), jnp.float32)
    z2 = jnp.zeros((NACC, D), jnp.float32)

    degp = _sc_deg(colp, z1)
    p0 = degp[0, :N].reshape(G, 1, RB)
    p1 = degp[1, :N].reshape(G, 1, RB)
    hp, dis2 = _tc_prep(p0, p1, x, W1)
    aggp = _sc_row_agg(hp, rowp, colp, z2)
    (sp2,) = _tc_mid(aggp[0], aggp[1], hp, dis2, b1.reshape(1, D), W2)
    aggsp = _sc_scalar_agg(sp2.reshape(N), rowp, colp, z1)
    q0 = aggsp[0, :N].reshape(G, 1, RB)
    q1 = aggsp[1, :N].reshape(G, 1, RB)
    (fin2,) = _tc_fin(q0, q1, sp2, dis2, b2.reshape(1, D))
    return fin2.reshape(N)


# speedup vs baseline: 1.0082x; 1.0082x over previous
import functools

import jax
import jax.numpy as jnp
from jax import lax
from jax.experimental import pallas as pl
from jax.experimental.pallas import tpu as pltpu
from jax.experimental.pallas import tpu_sc as plsc

N = 10000
E = 320000
D = 128
DH = D // 2
NC, NS, L = 2, 16, 16
NW = NC * NS
K = 128
CW = 80
EP = NW * CW * K
KR = 64
CWF = 320
NACC = 10240
RPT = NACC // NS
NACC_R = 10112
RPT_R = NACC_R // NS
DUMMY = N
RB = 400
G = N // RB

_mesh = plsc.VectorSubcoreMesh(core_axis_name="c", subcore_axis_name="s")
NBUF = 4
RBUF = 2
GG = 16


@functools.partial(
    pl.kernel,
    out_type=jax.ShapeDtypeStruct((NC, NACC), jnp.float32),
    mesh=_mesh,
    scratch_types=[
        pltpu.VMEM((CW, K), jnp.int32),
        pltpu.VMEM((K,), jnp.float32),
        pltpu.VMEM_SHARED((NACC,), jnp.float32),
        pltpu.SemaphoreType.DMA,
    ],
)
def _sc_deg(colp_hbm, z1_hbm, out_hbm, colb, oneb, acc, sem):
    cid = lax.axis_index("c")
    sid = lax.axis_index("s")
    wid = sid * NC + cid
    pltpu.sync_copy(z1_hbm.at[pl.ds(sid * RPT, RPT)],
                    acc.at[pl.ds(sid * RPT, RPT)])
    pltpu.sync_copy(colp_hbm.at[wid], colb)
    for t in range(K // L):
        oneb[pl.ds(t * L, L)] = jnp.ones((L,), jnp.float32)
    plsc.subcore_barrier()

    def fire(j, carry):
        pltpu.async_copy(oneb, acc.at[colb.at[j]], sem, add=True)
        return carry

    lax.fori_loop(0, CW, fire, 0)

    def drain(j, carry):
        pltpu.make_async_copy(oneb, acc.at[colb.at[j]], sem).wait()
        return carry

    lax.fori_loop(0, CW, drain, 0)
    plsc.subcore_barrier()
    pltpu.sync_copy(acc.at[pl.ds(sid * RPT, RPT)],
                    out_hbm.at[cid, pl.ds(sid * RPT, RPT)])


@functools.partial(
    pl.kernel,
    out_type=jax.ShapeDtypeStruct((NC, NACC), jnp.float32),
    mesh=_mesh,
    scratch_types=[
        pltpu.VMEM((CW, K), jnp.int32),
        pltpu.VMEM((CW, K), jnp.int32),
        pltpu.VMEM((K,), jnp.float32),
        pltpu.VMEM((K,), jnp.float32),
        pltpu.VMEM((K,), jnp.float32),
        pltpu.VMEM((K,), jnp.float32),
        pltpu.SemaphoreType.DMA,
        pltpu.SemaphoreType.DMA,
        pltpu.SemaphoreType.DMA,
        pltpu.SemaphoreType.DMA,
        pltpu.VMEM_SHARED((NACC,), jnp.float32),
    ],
)
def _sc_scalar_agg(sp_hbm, rowp_hbm, colp_hbm, z1_hbm, out_hbm,
                   rowb, colb, v0, v1, v2, v3, s0, s1, s2, s3, acc):
    valbs = (v0, v1, v2, v3)
    sems = (s0, s1, s2, s3)
    cid = lax.axis_index("c")
    sid = lax.axis_index("s")
    wid = sid * NC + cid
    pltpu.sync_copy(z1_hbm.at[pl.ds(sid * RPT, RPT)],
                    acc.at[pl.ds(sid * RPT, RPT)])
    pltpu.sync_copy(rowp_hbm.at[wid], rowb)
    pltpu.sync_copy(colp_hbm.at[wid], colb)
    plsc.subcore_barrier()

    for b in range(NBUF):
        pltpu.async_copy(sp_hbm.at[rowb.at[b]], valbs[b], sems[b])

    def step(i, carry):
        for b in range(NBUF):
            j = i * NBUF + b
            pltpu.make_async_copy(sp_hbm.at[rowb.at[j]], valbs[b],
                                  sems[b]).wait()
            pltpu.sync_copy(valbs[b], acc.at[colb.at[j]], add=True)
            jn = j + NBUF

            @pl.when(jn < CW)
            def _():
                pltpu.async_copy(sp_hbm.at[rowb.at[jn]], valbs[b], sems[b])
        return carry

    lax.fori_loop(0, CW // NBUF, step, 0)
    plsc.subcore_barrier()
    pltpu.sync_copy(acc.at[pl.ds(sid * RPT, RPT)],
                    out_hbm.at[cid, pl.ds(sid * RPT, RPT)])


@functools.partial(
    pl.kernel,
    out_type=jax.ShapeDtypeStruct((NC, NACC, D), jnp.float32),
    mesh=_mesh,
    scratch_types=[
        pltpu.VMEM((CW, K), jnp.int32),
        pltpu.VMEM((GG, K), jnp.int32),
        pltpu.VMEM((K, D), jnp.float32),
        pltpu.VMEM((K, D), jnp.float32),
        pltpu.SemaphoreType.DMA,
        pltpu.SemaphoreType.DMA,
        pltpu.VMEM_SHARED((NACC, D), jnp.float32),
    ],
)
def _sc_row_agg(hp_hbm, rowp_hbm, colp_hbm, z2_hbm, out_hbm,
                rowb, colg, m0, m1, s0, s1, acc):
    msgs = (m0, m1)
    sems = (s0, s1)
    cid = lax.axis_index("c")
    sid = lax.axis_index("s")
    wid = sid * NC + cid
    pltpu.sync_copy(z2_hbm.at[pl.ds(sid * RPT, RPT)],
                    acc.at[pl.ds(sid * RPT, RPT)])
    pltpu.sync_copy(rowp_hbm.at[wid], rowb)
    plsc.subcore_barrier()

    for b in range(RBUF):
        pltpu.async_copy(hp_hbm.at[rowb.at[b]], msgs[b], sems[b])

    def group(g, carry):
        pltpu.sync_copy(colp_hbm.at[wid, pl.ds(g * GG, GG)], colg)

        def step(i, carry2):
            for b in range(RBUF):
                jj = i * RBUF + b
                j = g * GG + jj
                pltpu.make_async_copy(hp_hbm.at[rowb.at[j]], msgs[b],
                                      sems[b]).wait()
                pltpu.sync_copy(msgs[b], acc.at[colg.at[jj]], add=True)
                jn = j + RBUF

                @pl.when(jn < CW)
                def _():
                    pltpu.async_copy(hp_hbm.at[rowb.at[jn]], msgs[b], sems[b])
            return carry2

        lax.fori_loop(0, GG // RBUF, step, 0)
        return carry

    lax.fori_loop(0, CW // GG, group, 0)
    plsc.subcore_barrier()
    pltpu.sync_copy(acc.at[pl.ds(sid * RPT, RPT)],
                    out_hbm.at[cid, pl.ds(sid * RPT, RPT)])


def _tc_prep_body(p0_ref, p1_ref, x_ref, w1_ref, hp_ref, dis_ref):
    deg = 1.0 + p0_ref[0, 0] + p1_ref[0, 0]
    dis = lax.rsqrt(deg)
    h = jnp.dot(x_ref[...], w1_ref[...], preferred_element_type=jnp.float32)
    hp_ref[...] = h * dis[:, None]
    dis_ref[0, 0] = dis


def _tc_mid_body(a0_ref, a1_ref, hp_ref, dis_ref, b1_ref, w2_ref, sp_ref):
    dis = dis_ref[0, 0]
    agg = a0_ref[...] + a1_ref[...] + hp_ref[...]
    out1 = agg * dis[:, None] + b1_ref[...]
    act = jnp.where(out1 >= 0, out1, 0.1 * out1)
    w2bar = jnp.mean(w2_ref[...], axis=1)
    s = jnp.sum(act * w2bar[None, :], axis=1)
    sp_ref[0, 0] = s * dis


def _tc_fin_body(q0_ref, q1_ref, sp_ref, dis_ref, b2_ref, out_ref):
    aggs = q0_ref[0, 0] + q1_ref[0, 0] + sp_ref[0, 0]
    out_ref[0, 0] = dis_ref[0, 0] * aggs + jnp.mean(b2_ref[...])


_blk_rb = pl.BlockSpec((1, 1, RB), lambda i: (i, 0, 0))
_blk_rows = pl.BlockSpec((RB, D), lambda i: (i, 0))
_blk_half = pl.BlockSpec((RB, DH), lambda i: (i, 0))
_blk_full = pl.BlockSpec((D, D), lambda i: (0, 0))
_blk_vec = pl.BlockSpec((1, D), lambda i: (0, 0))

_tc_prep = pl.pallas_call(
    _tc_prep_body,
    grid=(G,),
    in_specs=[_blk_rb, _blk_rb, _blk_rows, _blk_full],
    out_specs=[_blk_rows, _blk_rb],
    out_shape=[
        jax.ShapeDtypeStruct((N, D), jnp.float32),
        jax.ShapeDtypeStruct((G, 1, RB), jnp.float32),
    ],
)

_tc_mid = pl.pallas_call(
    _tc_mid_body,
    grid=(G,),
    in_specs=[_blk_rows, _blk_rows, _blk_rows, _blk_rb, _blk_vec, _blk_full],
    out_specs=[_blk_rb],
    out_shape=[jax.ShapeDtypeStruct((G, 1, RB), jnp.float32)],
)

_tc_fin = pl.pallas_call(
    _tc_fin_body,
    grid=(G,),
    in_specs=[_blk_rb, _blk_rb, _blk_rb, _blk_rb, _blk_vec],
    out_specs=[_blk_rb],
    out_shape=[jax.ShapeDtypeStruct((G, 1, RB), jnp.float32)],
)


def kernel(x, edge_index, W1, b1, W2, b2):
    row = edge_index[0]
    col = edge_index[1]
    row_p = jnp.concatenate([row, jnp.zeros((EP - E,), jnp.int32)])
    col_p = jnp.concatenate([col, jnp.full((EP - E,), DUMMY, jnp.int32)])
    rowp = row_p.reshape(NW, CW, K)
    colp = col_p.reshape(NW, CW, K)
    z1 = jnp.zeros((NACC,), jnp.float32)
    z2 = jnp.zeros((NACC, D), jnp.float32)

    degp = _sc_deg(colp, z1)
    p0 = degp[0, :N].reshape(G, 1, RB)
    p1 = degp[1, :N].reshape(G, 1, RB)
    hp, dis2 = _tc_prep(p0, p1, x, W1)
    aggp = _sc_row_agg(hp, rowp, colp, z2)
    (sp2,) = _tc_mid(aggp[0], aggp[1], hp, dis2, b1.reshape(1, D), W2)
    aggsp = _sc_scalar_agg(sp2.reshape(N), rowp, colp, z1)
    q0 = aggsp[0, :N].reshape(G, 1, RB)
    q1 = aggsp[1, :N].reshape(G, 1, RB)
    (fin2,) = _tc_fin(q0, q1, sp2, dis2, b2.reshape(1, D))
    return fin2.reshape(N)
